# baseline (speedup 1.0000x reference)
import jax
import jax.numpy as jnp
from jax.experimental import pallas as pl
from jax.experimental.pallas import tpu as pltpu

_N_E = 1024
_E_DIM = 64
_BETA = 0.25
_TOK = 16384
_TILE = 2048
_GRID = _TOK // _TILE


def _vq_body(zf_ref, emb_ref, esq_ref, loss_ref, zq_ref, enc_ref, idx_ref):
    i = pl.program_id(0)
    zf = zf_ref[...]
    emb = emb_ref[...]
    esq = esq_ref[...]
    zsq = jnp.sum(zf ** 2, axis=1, keepdims=True)

    mm = jax.lax.dot_general(
        zf, emb, (((1,), (1,)), ((), ())),
        preferred_element_type=jnp.float32)
    d = zsq + esq - 2.0 * mm

    dmin = jnp.min(d, axis=1, keepdims=True)
    col = jax.lax.broadcasted_iota(jnp.int32, d.shape, 1)
    idx = jnp.min(jnp.where(d == dmin, col, _N_E), axis=1)
    onehot = (col == idx[:, None]).astype(jnp.float32)
    enc_ref[...] = onehot
    idx_ref[...] = idx[:, None]

    zq = jax.lax.dot_general(
        onehot.astype(jnp.bfloat16), emb.astype(jnp.bfloat16),
        (((1,), (0,)), ((), ())),
        preferred_element_type=jnp.float32)
    zq_ref[...] = zf + (zq - zf)

    part = jnp.sum(dmin, keepdims=True)

    @pl.when(i == 0)
    def _init():
        loss_ref[...] = jnp.zeros((1, 1), jnp.float32)

    loss_ref[...] += part

    @pl.when(i == _GRID - 1)
    def _fin():
        loss_ref[...] = loss_ref[...] * ((1.0 + _BETA) / (_TOK * _E_DIM))


def _vq_call(zf, emb_weight, esq):
    return pl.pallas_call(
        _vq_body,
        grid=(_GRID,),
        in_specs=[
            pl.BlockSpec((_TILE, _E_DIM), lambda i: (i, 0)),
            pl.BlockSpec((_N_E, _E_DIM), lambda i: (0, 0)),
            pl.BlockSpec((1, _N_E), lambda i: (0, 0)),
        ],
        out_specs=[
            pl.BlockSpec((1, 1), lambda i: (0, 0)),
            pl.BlockSpec((_TILE, _E_DIM), lambda i: (i, 0)),
            pl.BlockSpec((_TILE, _N_E), lambda i: (i, 0)),
            pl.BlockSpec((_TILE, 1), lambda i: (i, 0)),
        ],
        out_shape=[
            jax.ShapeDtypeStruct((1, 1), jnp.float32),
            jax.ShapeDtypeStruct((_TOK, _E_DIM), jnp.float32),
            jax.ShapeDtypeStruct((_TOK, _N_E), jnp.float32),
            jax.ShapeDtypeStruct((_TOK, 1), jnp.int32),
        ],
        compiler_params=pltpu.CompilerParams(
            dimension_semantics=("arbitrary",)),
    )(zf, emb_weight, esq)


def kernel(z, emb_weight):
    zp = jnp.transpose(z, (0, 2, 3, 1))
    zf = zp.reshape(-1, _E_DIM)
    esq = jnp.sum(emb_weight ** 2, axis=1)[None, :]
    loss2, zq, enc, idx = _vq_call(zf, emb_weight, esq)
    z_q = jnp.transpose(zq.reshape(zp.shape), (0, 3, 1, 2))
    return (loss2[0, 0], z_q, enc, idx)


# speedup vs baseline: 1.0397x; 1.0397x over previous
import jax
import jax.numpy as jnp
from jax.experimental import pallas as pl
from jax.experimental.pallas import tpu as pltpu

_N_E = 1024
_E_DIM = 64
_BETA = 0.25
_TOK = 16384
_TILE = 2048
_GRID = _TOK // _TILE


def _vq_body(zf_ref, emb_ref, loss_ref, zq_ref, enc_ref, idx_ref):
    i = pl.program_id(0)
    zf = zf_ref[...]
    emb = emb_ref[...]
    esq = jnp.sum(emb ** 2, axis=1)[None, :]
    zsq = jnp.sum(zf ** 2, axis=1, keepdims=True)

    mm = jax.lax.dot_general(
        zf, emb, (((1,), (1,)), ((), ())),
        preferred_element_type=jnp.float32)
    d = zsq + esq - 2.0 * mm

    dmin = jnp.min(d, axis=1, keepdims=True)
    col = jax.lax.broadcasted_iota(jnp.int32, d.shape, 1)
    t = jnp.where(d == dmin, col, _N_E)
    idx = jnp.min(t, axis=1)
    onehot = (t == idx[:, None]).astype(jnp.float32)
    enc_ref[...] = onehot
    idx_ref[...] = idx[:, None]

    zq = jax.lax.dot_general(
        onehot, emb, (((1,), (0,)), ((), ())),
        preferred_element_type=jnp.float32)
    zq_ref[...] = zq

    part = jnp.sum(dmin, keepdims=True)

    @pl.when(i == 0)
    def _init():
        loss_ref[...] = jnp.zeros((1, 1), jnp.float32)

    loss_ref[...] += part

    @pl.when(i == _GRID - 1)
    def _fin():
        loss_ref[...] = loss_ref[...] * ((1.0 + _BETA) / (_TOK * _E_DIM))


def _vq_call(zf, emb_weight):
    return pl.pallas_call(
        _vq_body,
        grid=(_GRID,),
        in_specs=[
            pl.BlockSpec((_TILE, _E_DIM), lambda i: (i, 0)),
            pl.BlockSpec((_N_E, _E_DIM), lambda i: (0, 0)),
        ],
        out_specs=[
            pl.BlockSpec((1, 1), lambda i: (0, 0)),
            pl.BlockSpec((_TILE, _E_DIM), lambda i: (i, 0)),
            pl.BlockSpec((_TILE, _N_E), lambda i: (i, 0)),
            pl.BlockSpec((_TILE, 1), lambda i: (i, 0)),
        ],
        out_shape=[
            jax.ShapeDtypeStruct((1, 1), jnp.float32),
            jax.ShapeDtypeStruct((_TOK, _E_DIM), jnp.float32),
            jax.ShapeDtypeStruct((_TOK, _N_E), jnp.float32),
            jax.ShapeDtypeStruct((_TOK, 1), jnp.int32),
        ],
        compiler_params=pltpu.CompilerParams(
            dimension_semantics=("arbitrary",)),
    )(zf, emb_weight)


def kernel(z, emb_weight):
    zp = jnp.transpose(z, (0, 2, 3, 1))
    zf = zp.reshape(-1, _E_DIM)
    loss2, zq, enc, idx = _vq_call(zf, emb_weight)
    z_q = jnp.transpose(zq.reshape(zp.shape), (0, 3, 1, 2))
    return (loss2[0, 0], z_q, enc, idx)
